# Initial kernel scaffold; baseline (speedup 1.0000x reference)
import jax
import jax.numpy as jnp
from jax.experimental import pallas as pl


def kernel(word, pos, W_word, W_pos):
    raise NotImplementedError("write your pallas kernel here")


# speedup vs baseline: 5.8192x; 5.8192x over previous
import jax
import jax.numpy as jnp
from jax import lax
from jax.experimental import pallas as pl
from jax.experimental.pallas import tpu as pltpu
from jax.experimental.pallas import tpu_sc as plsc

BATCH = 4096
MAX_LEN = 200
WORD_DIM = 64
POS_DIM = 16
OUT_DIM = WORD_DIM + POS_DIM
N = BATCH * MAX_LEN
STEP = 128
NROWS = N // STEP
NW = 32
ROWS_PER_W = NROWS // NW
K = 8
NCHUNK = ROWS_PER_W // K


def _sc_body(word_hbm, pos_hbm, ww_hbm, wp_hbm, out_hbm,
             idx_w, idx_p, rows_w, rows_p, sem_g, sem_w):
    wid = lax.axis_index("s") * 2 + lax.axis_index("c")
    row0 = wid * ROWS_PER_W

    def chunk(ci, carry):
        r = row0 + ci * K
        pltpu.sync_copy(word_hbm.at[pl.ds(r, K)], idx_w)
        pltpu.sync_copy(pos_hbm.at[pl.ds(r, K)], idx_p)
        gathers = []
        for j in range(K):
            gathers.append(
                pltpu.async_copy(ww_hbm.at[idx_w.at[j]], rows_w.at[j], sem_g))
            gathers.append(
                pltpu.async_copy(wp_hbm.at[idx_p.at[j]], rows_p.at[j], sem_g))
        for g in gathers:
            g.wait()
        writes = []
        for j in range(K):
            o = (r + j) * STEP
            writes.append(pltpu.async_copy(
                rows_w.at[j],
                out_hbm.at[pl.ds(o, STEP), pl.ds(0, WORD_DIM)], sem_w))
            writes.append(pltpu.async_copy(
                rows_p.at[j],
                out_hbm.at[pl.ds(o, STEP), pl.ds(WORD_DIM, POS_DIM)], sem_w))
        for w in writes:
            w.wait()
        return carry

    lax.fori_loop(0, NCHUNK, chunk, 0)


def kernel(word, pos, W_word, W_pos):
    word2 = word.reshape(NROWS, STEP).astype(jnp.int32)
    pos2 = pos.reshape(NROWS, STEP).astype(jnp.int32)
    mesh = plsc.VectorSubcoreMesh(core_axis_name="c", subcore_axis_name="s")
    out = pl.kernel(
        _sc_body,
        mesh=mesh,
        out_type=jax.ShapeDtypeStruct((N, OUT_DIM), jnp.float32),
        compiler_params=pltpu.CompilerParams(use_tc_tiling_on_sc=False),
        scratch_types=[
            pltpu.VMEM((K, STEP), jnp.int32),
            pltpu.VMEM((K, STEP), jnp.int32),
            pltpu.VMEM((K, STEP, WORD_DIM), jnp.float32),
            pltpu.VMEM((K, STEP, POS_DIM), jnp.float32),
            pltpu.SemaphoreType.DMA,
            pltpu.SemaphoreType.DMA,
        ],
    )(word2, pos2, W_word, W_pos)
    return out.reshape(BATCH, MAX_LEN, OUT_DIM)


# speedup vs baseline: 5.9238x; 1.0180x over previous
import jax
import jax.numpy as jnp
from jax import lax
from jax.experimental import pallas as pl
from jax.experimental.pallas import tpu as pltpu
from jax.experimental.pallas import tpu_sc as plsc

BATCH = 4096
MAX_LEN = 200
WORD_DIM = 64
POS_DIM = 16
OUT_DIM = WORD_DIM + POS_DIM
N = BATCH * MAX_LEN
STEP = 128
NROWS = N // STEP
NW = 32
ROWS_PER_W = NROWS // NW
K = 5
NPAIR = ROWS_PER_W // (2 * K)


def _sc_body(word_hbm, pos_hbm, ww_hbm, wp_hbm, out_hbm,
             iw0, ip0, rw0, rp0, iw1, ip1, rw1, rp1,
             sg0, sw0, sg1, sw1):
    sets = ((iw0, ip0, rw0, rp0, sg0, sw0),
            (iw1, ip1, rw1, rp1, sg1, sw1))
    wid = lax.axis_index("s") * 2 + lax.axis_index("c")
    row0 = wid * ROWS_PER_W

    def fire_gathers(r, s):
        iw, ip, rw, rp, sg, _ = s
        pltpu.sync_copy(word_hbm.at[pl.ds(r, K)], iw)
        pltpu.sync_copy(pos_hbm.at[pl.ds(r, K)], ip)
        gs = []
        for j in range(K):
            gs.append(pltpu.async_copy(ww_hbm.at[iw.at[j]], rw.at[j], sg))
            gs.append(pltpu.async_copy(wp_hbm.at[ip.at[j]], rp.at[j], sg))
        return gs

    def fire_writes(r, s):
        _, _, rw, rp, _, sw = s
        ws = []
        for j in range(K):
            o = (r + j) * STEP
            ws.append(pltpu.async_copy(
                rw.at[j], out_hbm.at[pl.ds(o, STEP), pl.ds(0, WORD_DIM)], sw))
            ws.append(pltpu.async_copy(
                rp.at[j],
                out_hbm.at[pl.ds(o, STEP), pl.ds(WORD_DIM, POS_DIM)], sw))
        return ws

    def pair(pi, carry):
        r_a = row0 + pi * 2 * K
        r_b = r_a + K
        g_a = fire_gathers(r_a, sets[0])
        g_b = fire_gathers(r_b, sets[1])
        for g in g_a:
            g.wait()
        w_a = fire_writes(r_a, sets[0])
        for g in g_b:
            g.wait()
        w_b = fire_writes(r_b, sets[1])
        for w in w_a:
            w.wait()
        for w in w_b:
            w.wait()
        return carry

    lax.fori_loop(0, NPAIR, pair, 0)


def kernel(word, pos, W_word, W_pos):
    word2 = word.reshape(NROWS, STEP).astype(jnp.int32)
    pos2 = pos.reshape(NROWS, STEP).astype(jnp.int32)
    mesh = plsc.VectorSubcoreMesh(core_axis_name="c", subcore_axis_name="s")
    out = pl.kernel(
        _sc_body,
        mesh=mesh,
        out_type=jax.ShapeDtypeStruct((N, OUT_DIM), jnp.float32),
        compiler_params=pltpu.CompilerParams(use_tc_tiling_on_sc=False),
        scratch_types=[
            pltpu.VMEM((K, STEP), jnp.int32),
            pltpu.VMEM((K, STEP), jnp.int32),
            pltpu.VMEM((K, STEP, WORD_DIM), jnp.float32),
            pltpu.VMEM((K, STEP, POS_DIM), jnp.float32),
            pltpu.VMEM((K, STEP), jnp.int32),
            pltpu.VMEM((K, STEP), jnp.int32),
            pltpu.VMEM((K, STEP, WORD_DIM), jnp.float32),
            pltpu.VMEM((K, STEP, POS_DIM), jnp.float32),
            pltpu.SemaphoreType.DMA,
            pltpu.SemaphoreType.DMA,
            pltpu.SemaphoreType.DMA,
            pltpu.SemaphoreType.DMA,
        ],
    )(word2, pos2, W_word, W_pos)
    return out.reshape(BATCH, MAX_LEN, OUT_DIM)
